# baseline (speedup 1.0000x reference)
import dataclasses
import functools

import jax
import jax.numpy as jnp
from jax import lax
from jax.experimental import pallas as pl
from jax.experimental.pallas import tpu as pltpu
from jax.experimental.pallas import tpu_sc as plsc

N = 10000
E = 320000
D = 128

NC = 2
NS = 16
L = 16

EDGES_PER_TILE = E // (NC * NS)
C = 80
NCHUNK = EDGES_PER_TILE // C
FLUSH_A = 632
FLUSH_B = N - FLUSH_A * (NS - 1)
DEN_PAD = 10240

ROW_BLK = 400

_SC_PARAMS = pltpu.CompilerParams()
if "needs_layout_passes" in pltpu.CompilerParams.__dataclass_fields__:
    _SC_PARAMS = dataclasses.replace(_SC_PARAMS, needs_layout_passes=False)


def _dense_body(x_ref, w_ref, al_ref, ar_ref, h_ref, el_ref, er_ref):
    h = jnp.dot(x_ref[...], w_ref[...], preferred_element_type=jnp.float32)
    h_ref[...] = h
    el_ref[...] = jnp.dot(h, al_ref[...], preferred_element_type=jnp.float32)
    er_ref[...] = jnp.dot(h, ar_ref[...], preferred_element_type=jnp.float32)


def _dense(features, fc_weight, al, ar):
    return pl.pallas_call(
        _dense_body,
        grid=(N // ROW_BLK,),
        in_specs=[
            pl.BlockSpec((ROW_BLK, D), lambda i: (i, 0)),
            pl.BlockSpec((D, D), lambda i: (0, 0)),
            pl.BlockSpec((D, 1), lambda i: (0, 0)),
            pl.BlockSpec((D, 1), lambda i: (0, 0)),
        ],
        out_specs=[
            pl.BlockSpec((ROW_BLK, D), lambda i: (i, 0)),
            pl.BlockSpec((ROW_BLK, 1), lambda i: (i, 0)),
            pl.BlockSpec((ROW_BLK, 1), lambda i: (i, 0)),
        ],
        out_shape=[
            jax.ShapeDtypeStruct((N, D), jnp.float32),
            jax.ShapeDtypeStruct((N, 1), jnp.float32),
            jax.ShapeDtypeStruct((N, 1), jnp.float32),
        ],
    )(features, fc_weight, al, ar)


@functools.partial(
    pl.kernel,
    out_type=(
        jax.ShapeDtypeStruct((NC, N, D), jnp.float32),
        jax.ShapeDtypeStruct((NC, DEN_PAD), jnp.float32),
    ),
    mesh=plsc.VectorSubcoreMesh(core_axis_name="c", subcore_axis_name="s"),
    compiler_params=_SC_PARAMS,
    scratch_types=[
        pltpu.VMEM((N,), jnp.float32),
        pltpu.VMEM((N,), jnp.float32),
        pltpu.VMEM((C,), jnp.int32),
        pltpu.VMEM((C,), jnp.int32),
        pltpu.VMEM((C,), jnp.int32),
        pltpu.VMEM((C,), jnp.int32),
        pltpu.VMEM((C, D), jnp.float32),
        pltpu.VMEM((C, D), jnp.float32),
        pltpu.VMEM((C,), jnp.float32),
        pltpu.VMEM((C,), jnp.float32),
        pltpu.VMEM((C,), jnp.int32),
        pltpu.VMEM((C,), jnp.int32),
        pltpu.VMEM((640,), jnp.float32),
        pltpu.VMEM_SHARED((N, D), jnp.float32),
        pltpu.VMEM_SHARED((DEN_PAD,), jnp.float32),
        pltpu.SemaphoreType.DMA,
        pltpu.SemaphoreType.DMA,
        pltpu.SemaphoreType.DMA,
        pltpu.SemaphoreType.DMA,
        pltpu.SemaphoreType.DMA,
        pltpu.SemaphoreType.DMA,
        pltpu.SemaphoreType.DMA,
        pltpu.SemaphoreType.DMA,
        pltpu.SemaphoreType.DMA,
        pltpu.SemaphoreType.DMA,
        pltpu.SemaphoreType.DMA,
        pltpu.SemaphoreType.DMA,
        pltpu.SemaphoreType.DMA,
        pltpu.SemaphoreType.DMA,
        pltpu.SemaphoreType.DMA,
    ],
)
def _sc_edges(h_hbm, el_hbm, er_hbm, ei_hbm, acc_hbm, den_hbm,
              el_v, er_v, si0, si1, di0, di1, rows0, rows1, w0, w1, db0, db1,
              z_v, acc_sh, den_sh,
              sem_pre, semi0, semi1,
              mg00, mg01, mg02, mg03, mg04,
              mg10, mg11, mg12, mg13, mg14,
              sems0, sems1):
    c = lax.axis_index("c")
    s = lax.axis_index("s")
    wid = c * NS + s
    tbase = pl.multiple_of(wid * EDGES_PER_TILE, 8)

    si = (si0, si1)
    di = (di0, di1)
    rows = (rows0, rows1)
    wb = (w0, w1)
    db = (db0, db1)
    semi = (semi0, semi1)
    semg = ((mg00, mg01, mg02, mg03, mg04),
            (mg10, mg11, mg12, mg13, mg14))
    sems = (sems0, sems1)

    pltpu.async_copy(el_hbm, el_v, sem_pre)
    pltpu.async_copy(er_hbm, er_v, sem_pre)

    def issue_idx(g, b):
        ebase = pl.multiple_of(tbase + g * C, 8)
        pltpu.async_copy(ei_hbm.at[pl.ds(ebase, C)], si[b], semi[b])
        pltpu.async_copy(ei_hbm.at[pl.ds(E + ebase, C)], di[b], semi[b])

    def wait_idx(b):
        pltpu.make_async_copy(ei_hbm.at[pl.ds(0, C)], si[b], semi[b]).wait()
        pltpu.make_async_copy(ei_hbm.at[pl.ds(0, C)], di[b], semi[b]).wait()

    def issue_gather(b):
        for k in range(C // L):
            sv = si[b][pl.ds(k * L, L)]
            pltpu.async_copy(h_hbm.at[sv],
                             rows[b].at[pl.ds(k * L, L)], semg[b][k])

    def wait_scatter(b):
        pltpu.make_async_copy(rows[b], acc_sh.at[db[b]], sems[b]).wait()
        pltpu.make_async_copy(wb[b], den_sh.at[db[b]], sems[b]).wait()

    def streams(b):
        for k in range(C // L):
            sv = si[b][pl.ds(k * L, L)]
            dv = di[b][pl.ds(k * L, L)]
            e = plsc.load_gather(el_v, [sv]) + plsc.load_gather(er_v, [dv])
            e = jnp.where(e > 0.0, e, 0.2 * e)
            wb[b][pl.ds(k * L, L)] = jnp.exp(e)
            db[b][pl.ds(k * L, L)] = dv

            pltpu.make_async_copy(h_hbm.at[sv],
                                  rows[b].at[pl.ds(k * L, L)],
                                  semg[b][k]).wait()

            @pl.loop(0, L)
            def _(i):
                iv = jnp.full((L,), i, dtype=jnp.int32) + (k * L)
                wv = plsc.load_gather(wb[b], [iv])
                r = k * L + i
                for j in range(D // L):
                    rows[b][r, pl.ds(j * L, L)] = (
                        rows[b][r, pl.ds(j * L, L)] * wv)

            pltpu.async_copy(rows[b].at[pl.ds(k * L, L)],
                             acc_sh.at[dv], sems[b], add=True)

        pltpu.async_copy(wb[b], den_sh.at[db[b]], sems[b], add=True)

    issue_idx(0, 0)
    issue_idx(1, 1)

    zeros = jnp.zeros((L,), jnp.float32)

    @pl.loop(0, 640, step=L)
    def _(i):
        z_v[pl.ds(i, L)] = zeros

    @pl.loop(0, C)
    def _(i):
        for j in range(D // L):
            rows0[i, pl.ds(j * L, L)] = zeros

    rbase = pl.multiple_of(s * FLUSH_A, 8)

    def _zero_acc(base, nrows):
        for k in range(nrows // C):
            pltpu.sync_copy(rows0, acc_sh.at[pl.ds(base + k * C, C)])
        rem = nrows % C
        if rem:
            pltpu.sync_copy(rows0.at[pl.ds(0, rem)],
                            acc_sh.at[pl.ds(base + nrows - rem, rem)])

    @pl.when(s != NS - 1)
    def _():
        _zero_acc(rbase, FLUSH_A)

    @pl.when(s == NS - 1)
    def _():
        _zero_acc((NS - 1) * FLUSH_A, FLUSH_B)

    dbase = pl.multiple_of(s * 640, 8)
    pltpu.sync_copy(z_v, den_sh.at[pl.ds(dbase, 640)])

    pltpu.make_async_copy(el_hbm, el_v, sem_pre).wait()
    pltpu.make_async_copy(er_hbm, er_v, sem_pre).wait()

    wait_idx(0)
    issue_gather(0)
    plsc.subcore_barrier()

    @pl.loop(0, (NCHUNK - 1) // 2)
    def _(p):
        for b in range(2):
            g = 2 * p + b

            @pl.when(p + b > 0)
            def _():
                wait_scatter(1 - b)

            wait_idx(1 - b)
            issue_gather(1 - b)
            streams(b)

            @pl.when(g + 2 < NCHUNK)
            def _():
                issue_idx(g + 2, b)

    wait_scatter(1)
    streams(0)
    wait_scatter(0)

    plsc.subcore_barrier()

    @pl.when(s != NS - 1)
    def _():
        pltpu.sync_copy(acc_sh.at[pl.ds(rbase, FLUSH_A)],
                        acc_hbm.at[c, pl.ds(rbase, FLUSH_A)])

    @pl.when(s == NS - 1)
    def _():
        pltpu.sync_copy(acc_sh.at[pl.ds((NS - 1) * FLUSH_A, FLUSH_B)],
                        acc_hbm.at[c, pl.ds((NS - 1) * FLUSH_A, FLUSH_B)])

    pltpu.sync_copy(den_sh.at[pl.ds(dbase, 640)],
                    den_hbm.at[c, pl.ds(dbase, 640)])


def _final_body(acc_ref, den_ref, out_ref):
    a = acc_ref[0] + acc_ref[1]
    dsum = den_ref[0] + den_ref[1]
    out_ref[...] = jnp.where(dsum == 0.0, 0.0,
                             a / jnp.where(dsum == 0.0, 1.0, dsum))


def _finalize(acc, den):
    return pl.pallas_call(
        _final_body,
        grid=(N // ROW_BLK,),
        in_specs=[
            pl.BlockSpec((NC, ROW_BLK, D), lambda i: (0, i, 0)),
            pl.BlockSpec((NC, ROW_BLK, 1), lambda i: (0, i, 0)),
        ],
        out_specs=pl.BlockSpec((ROW_BLK, D), lambda i: (i, 0)),
        out_shape=jax.ShapeDtypeStruct((N, D), jnp.float32),
    )(acc, den)


def kernel(features, edge_index, fc_weight, attn_l, attn_r):
    h, el, er = _dense(features, fc_weight, attn_l.T, attn_r.T)
    acc, den = _sc_edges(h, el.reshape(N), er.reshape(N),
                         edge_index.reshape(2 * E))
    return _finalize(acc, den.reshape(NC, DEN_PAD, 1))


# speedup vs baseline: 1.0220x; 1.0220x over previous
import dataclasses
import functools

import jax
import jax.numpy as jnp
from jax import lax
from jax.experimental import pallas as pl
from jax.experimental.pallas import tpu as pltpu
from jax.experimental.pallas import tpu_sc as plsc

N = 10000
E = 320000
D = 128

NC = 2
NS = 16
L = 16

EDGES_PER_TILE = E // (NC * NS)
C = 80
NCHUNK = EDGES_PER_TILE // C
FLUSH_A = 632
FLUSH_B = N - FLUSH_A * (NS - 1)
DEN_PAD = 10240

ROW_BLK = 400

_SC_PARAMS = pltpu.CompilerParams()
if "needs_layout_passes" in pltpu.CompilerParams.__dataclass_fields__:
    _SC_PARAMS = dataclasses.replace(_SC_PARAMS, needs_layout_passes=False)


def _dense_body(x_ref, w_ref, al_ref, ar_ref, h_ref, el_ref, er_ref):
    h = jnp.dot(x_ref[...], w_ref[...], preferred_element_type=jnp.float32)
    h_ref[...] = h
    el_ref[...] = jnp.dot(h, al_ref[...], preferred_element_type=jnp.float32)
    er_ref[...] = jnp.dot(h, ar_ref[...], preferred_element_type=jnp.float32)


def _dense(features, fc_weight, al, ar):
    return pl.pallas_call(
        _dense_body,
        grid=(N // ROW_BLK,),
        in_specs=[
            pl.BlockSpec((ROW_BLK, D), lambda i: (i, 0)),
            pl.BlockSpec((D, D), lambda i: (0, 0)),
            pl.BlockSpec((D, 1), lambda i: (0, 0)),
            pl.BlockSpec((D, 1), lambda i: (0, 0)),
        ],
        out_specs=[
            pl.BlockSpec((ROW_BLK, D), lambda i: (i, 0)),
            pl.BlockSpec((ROW_BLK, 1), lambda i: (i, 0)),
            pl.BlockSpec((ROW_BLK, 1), lambda i: (i, 0)),
        ],
        out_shape=[
            jax.ShapeDtypeStruct((N, D), jnp.float32),
            jax.ShapeDtypeStruct((N, 1), jnp.float32),
            jax.ShapeDtypeStruct((N, 1), jnp.float32),
        ],
    )(features, fc_weight, al, ar)


@functools.partial(
    pl.kernel,
    out_type=(
        jax.ShapeDtypeStruct((NC, N, D), jnp.float32),
        jax.ShapeDtypeStruct((NC, DEN_PAD), jnp.float32),
    ),
    mesh=plsc.VectorSubcoreMesh(core_axis_name="c", subcore_axis_name="s"),
    compiler_params=_SC_PARAMS,
    scratch_types=[
        pltpu.VMEM((N,), jnp.float32),
        pltpu.VMEM((N,), jnp.float32),
        pltpu.VMEM((C,), jnp.int32),
        pltpu.VMEM((C,), jnp.int32),
        pltpu.VMEM((C,), jnp.int32),
        pltpu.VMEM((C,), jnp.int32),
        pltpu.VMEM((C, D), jnp.float32),
        pltpu.VMEM((C, D), jnp.float32),
        pltpu.VMEM((C,), jnp.float32),
        pltpu.VMEM((C,), jnp.float32),
        pltpu.VMEM((C,), jnp.int32),
        pltpu.VMEM((C,), jnp.int32),
        pltpu.VMEM((640,), jnp.float32),
        pltpu.VMEM_SHARED((N, D), jnp.float32),
        pltpu.VMEM_SHARED((DEN_PAD,), jnp.float32),
        pltpu.SemaphoreType.DMA,
        pltpu.SemaphoreType.DMA,
        pltpu.SemaphoreType.DMA,
        pltpu.SemaphoreType.DMA,
        pltpu.SemaphoreType.DMA,
        pltpu.SemaphoreType.DMA,
        pltpu.SemaphoreType.DMA,
    ],
)
def _sc_edges(h_hbm, el_hbm, er_hbm, ei_hbm, acc_hbm, den_hbm,
              el_v, er_v, si0, si1, di0, di1, rows0, rows1, w0, w1, db0, db1,
              z_v, acc_sh, den_sh,
              sem_pre, semi0, semi1, semg0, semg1, sems0, sems1):
    c = lax.axis_index("c")
    s = lax.axis_index("s")
    wid = c * NS + s
    tbase = pl.multiple_of(wid * EDGES_PER_TILE, 8)

    si = (si0, si1)
    di = (di0, di1)
    rows = (rows0, rows1)
    wb = (w0, w1)
    db = (db0, db1)
    semi = (semi0, semi1)
    semg = (semg0, semg1)
    sems = (sems0, sems1)

    pltpu.async_copy(el_hbm, el_v, sem_pre)
    pltpu.async_copy(er_hbm, er_v, sem_pre)

    def issue_idx(g, b):
        ebase = pl.multiple_of(tbase + g * C, 8)
        pltpu.async_copy(ei_hbm.at[pl.ds(ebase, C)], si[b], semi[b])
        pltpu.async_copy(ei_hbm.at[pl.ds(E + ebase, C)], di[b], semi[b])

    def wait_idx(b):
        pltpu.make_async_copy(ei_hbm.at[pl.ds(0, C)], si[b], semi[b]).wait()
        pltpu.make_async_copy(ei_hbm.at[pl.ds(0, C)], di[b], semi[b]).wait()

    def issue_gather(b):
        pltpu.async_copy(h_hbm.at[si[b]], rows[b], semg[b])

    def wait_gather(b):
        pltpu.make_async_copy(h_hbm.at[si[b]], rows[b], semg[b]).wait()

    def issue_scatter(b):
        pltpu.async_copy(rows[b], acc_sh.at[db[b]], sems[b], add=True)
        pltpu.async_copy(wb[b], den_sh.at[db[b]], sems[b], add=True)

    def wait_scatter(b):
        pltpu.make_async_copy(rows[b], acc_sh.at[db[b]], sems[b]).wait()
        pltpu.make_async_copy(wb[b], den_sh.at[db[b]], sems[b]).wait()

    def compute(b):
        for vi in range(C // L):
            sv = si[b][pl.ds(vi * L, L)]
            dv = di[b][pl.ds(vi * L, L)]
            e = plsc.load_gather(el_v, [sv]) + plsc.load_gather(er_v, [dv])
            e = jnp.where(e > 0.0, e, 0.2 * e)
            wb[b][pl.ds(vi * L, L)] = jnp.exp(e)
            db[b][pl.ds(vi * L, L)] = dv

        @pl.loop(0, C)
        def _(i):
            iv = jnp.full((L,), i, dtype=jnp.int32)
            wv = plsc.load_gather(wb[b], [iv])
            for j in range(D // L):
                rows[b][i, pl.ds(j * L, L)] = rows[b][i, pl.ds(j * L, L)] * wv

    issue_idx(0, 0)
    issue_idx(1, 1)

    zeros = jnp.zeros((L,), jnp.float32)

    @pl.loop(0, 640, step=L)
    def _(i):
        z_v[pl.ds(i, L)] = zeros

    @pl.loop(0, C)
    def _(i):
        for j in range(D // L):
            rows0[i, pl.ds(j * L, L)] = zeros

    rbase = pl.multiple_of(s * FLUSH_A, 8)

    def _zero_acc(base, nrows):
        for k in range(nrows // C):
            pltpu.sync_copy(rows0, acc_sh.at[pl.ds(base + k * C, C)])
        rem = nrows % C
        if rem:
            pltpu.sync_copy(rows0.at[pl.ds(0, rem)],
                            acc_sh.at[pl.ds(base + nrows - rem, rem)])

    @pl.when(s != NS - 1)
    def _():
        _zero_acc(rbase, FLUSH_A)

    @pl.when(s == NS - 1)
    def _():
        _zero_acc((NS - 1) * FLUSH_A, FLUSH_B)

    dbase = pl.multiple_of(s * 640, 8)
    pltpu.sync_copy(z_v, den_sh.at[pl.ds(dbase, 640)])

    pltpu.make_async_copy(el_hbm, el_v, sem_pre).wait()
    pltpu.make_async_copy(er_hbm, er_v, sem_pre).wait()

    wait_idx(0)
    issue_gather(0)
    plsc.subcore_barrier()

    @pl.loop(0, (NCHUNK - 1) // 2)
    def _(p):
        for b in range(2):
            g = 2 * p + b

            @pl.when(p + b > 0)
            def _():
                wait_scatter(1 - b)

            wait_idx(1 - b)
            issue_gather(1 - b)
            wait_gather(b)
            compute(b)

            @pl.when(g + 2 < NCHUNK)
            def _():
                issue_idx(g + 2, b)

            issue_scatter(b)

    wait_gather(0)
    compute(0)
    wait_scatter(1)
    issue_scatter(0)
    wait_scatter(0)

    plsc.subcore_barrier()

    @pl.when(s != NS - 1)
    def _():
        pltpu.sync_copy(acc_sh.at[pl.ds(rbase, FLUSH_A)],
                        acc_hbm.at[c, pl.ds(rbase, FLUSH_A)])

    @pl.when(s == NS - 1)
    def _():
        pltpu.sync_copy(acc_sh.at[pl.ds((NS - 1) * FLUSH_A, FLUSH_B)],
                        acc_hbm.at[c, pl.ds((NS - 1) * FLUSH_A, FLUSH_B)])

    pltpu.sync_copy(den_sh.at[pl.ds(dbase, 640)],
                    den_hbm.at[c, pl.ds(dbase, 640)])


def _final_body(acc_ref, den_ref, out_ref):
    a = acc_ref[0] + acc_ref[1]
    dsum = den_ref[0] + den_ref[1]
    out_ref[...] = jnp.where(dsum == 0.0, 0.0,
                             a / jnp.where(dsum == 0.0, 1.0, dsum))


def _finalize(acc, den):
    return pl.pallas_call(
        _final_body,
        grid=(N // ROW_BLK,),
        in_specs=[
            pl.BlockSpec((NC, ROW_BLK, D), lambda i: (0, i, 0)),
            pl.BlockSpec((NC, ROW_BLK, 1), lambda i: (0, i, 0)),
        ],
        out_specs=pl.BlockSpec((ROW_BLK, D), lambda i: (i, 0)),
        out_shape=jax.ShapeDtypeStruct((N, D), jnp.float32),
    )(acc, den)


def kernel(features, edge_index, fc_weight, attn_l, attn_r):
    h, el, er = _dense(features, fc_weight, attn_l.T, attn_r.T)
    acc, den = _sc_edges(h, el.reshape(N), er.reshape(N),
                         edge_index.reshape(2 * E))
    return _finalize(acc, den.reshape(NC, DEN_PAD, 1))


# speedup vs baseline: 1.0257x; 1.0036x over previous
import dataclasses
import functools

import jax
import jax.numpy as jnp
from jax import lax
from jax.experimental import pallas as pl
from jax.experimental.pallas import tpu as pltpu
from jax.experimental.pallas import tpu_sc as plsc

N = 10000
E = 320000
D = 128

NC = 2
NS = 16
L = 16

EDGES_PER_TILE = E // (NC * NS)
C = 80
NCHUNK = EDGES_PER_TILE // C
FLUSH_A = 632
FLUSH_B = N - FLUSH_A * (NS - 1)
DEN_PAD = 10240

ROW_BLK = 400

_SC_PARAMS = pltpu.CompilerParams()
if "needs_layout_passes" in pltpu.CompilerParams.__dataclass_fields__:
    _SC_PARAMS = dataclasses.replace(_SC_PARAMS, needs_layout_passes=False)


def _dense_body(x_ref, w_ref, al_ref, ar_ref, h_ref, el_ref, er_ref):
    h = jnp.dot(x_ref[...], w_ref[...], preferred_element_type=jnp.float32)
    h_ref[...] = h
    el_ref[...] = jnp.dot(h, al_ref[...], preferred_element_type=jnp.float32)
    er_ref[...] = jnp.dot(h, ar_ref[...], preferred_element_type=jnp.float32)


def _dense(features, fc_weight, al, ar):
    return pl.pallas_call(
        _dense_body,
        grid=(N // ROW_BLK,),
        in_specs=[
            pl.BlockSpec((ROW_BLK, D), lambda i: (i, 0)),
            pl.BlockSpec((D, D), lambda i: (0, 0)),
            pl.BlockSpec((D, 1), lambda i: (0, 0)),
            pl.BlockSpec((D, 1), lambda i: (0, 0)),
        ],
        out_specs=[
            pl.BlockSpec((ROW_BLK, D), lambda i: (i, 0)),
            pl.BlockSpec((ROW_BLK, 1), lambda i: (i, 0)),
            pl.BlockSpec((ROW_BLK, 1), lambda i: (i, 0)),
        ],
        out_shape=[
            jax.ShapeDtypeStruct((N, D), jnp.float32),
            jax.ShapeDtypeStruct((N, 1), jnp.float32),
            jax.ShapeDtypeStruct((N, 1), jnp.float32),
        ],
    )(features, fc_weight, al, ar)


@functools.partial(
    pl.kernel,
    out_type=(
        jax.ShapeDtypeStruct((NC, N, D), jnp.float32),
        jax.ShapeDtypeStruct((NC, DEN_PAD), jnp.float32),
    ),
    mesh=plsc.VectorSubcoreMesh(core_axis_name="c", subcore_axis_name="s"),
    compiler_params=_SC_PARAMS,
    scratch_types=[
        pltpu.VMEM((N,), jnp.float32),
        pltpu.VMEM((N,), jnp.float32),
        pltpu.VMEM((C,), jnp.int32),
        pltpu.VMEM((C,), jnp.int32),
        pltpu.VMEM((C,), jnp.int32),
        pltpu.VMEM((C,), jnp.int32),
        pltpu.VMEM((C, D), jnp.float32),
        pltpu.VMEM((C, D), jnp.float32),
        pltpu.VMEM((C,), jnp.float32),
        pltpu.VMEM((C,), jnp.float32),
        pltpu.VMEM((C,), jnp.int32),
        pltpu.VMEM((C,), jnp.int32),
        pltpu.VMEM((640,), jnp.float32),
        pltpu.VMEM_SHARED((N, D), jnp.float32),
        pltpu.VMEM_SHARED((DEN_PAD,), jnp.float32),
        pltpu.SemaphoreType.DMA,
        pltpu.SemaphoreType.DMA,
        pltpu.SemaphoreType.DMA,
        pltpu.SemaphoreType.DMA,
        pltpu.SemaphoreType.DMA,
        pltpu.SemaphoreType.DMA,
        pltpu.SemaphoreType.DMA,
    ],
)
def _sc_edges(h_hbm, el_hbm, er_hbm, ei_hbm, acc_hbm, den_hbm,
              el_v, er_v, si0, si1, di0, di1, rows0, rows1, w0, w1, db0, db1,
              z_v, acc_sh, den_sh,
              sem_pre, semi0, semi1, semg0, semg1, sems0, sems1):
    c = lax.axis_index("c")
    s = lax.axis_index("s")
    wid = c * NS + s
    tbase = pl.multiple_of(wid * EDGES_PER_TILE, 8)

    si = (si0, si1)
    di = (di0, di1)
    rows = (rows0, rows1)
    wb = (w0, w1)
    db = (db0, db1)
    semi = (semi0, semi1)
    semg = (semg0, semg1)
    sems = (sems0, sems1)

    pltpu.async_copy(el_hbm, el_v, sem_pre)
    pltpu.async_copy(er_hbm, er_v, sem_pre)

    def issue_idx(g, b):
        ebase = pl.multiple_of(tbase + g * C, 8)
        pltpu.async_copy(ei_hbm.at[pl.ds(ebase, C)], si[b], semi[b])
        pltpu.async_copy(ei_hbm.at[pl.ds(E + ebase, C)], di[b], semi[b])

    def wait_idx(b):
        pltpu.make_async_copy(ei_hbm.at[pl.ds(0, C)], si[b], semi[b]).wait()
        pltpu.make_async_copy(ei_hbm.at[pl.ds(0, C)], di[b], semi[b]).wait()

    def issue_gather(b):
        pltpu.async_copy(h_hbm.at[si[b]], rows[b], semg[b])

    def wait_gather(b):
        pltpu.make_async_copy(h_hbm.at[si[b]], rows[b], semg[b]).wait()

    def issue_scatter(b):
        pltpu.async_copy(rows[b], acc_sh.at[db[b]], sems[b], add=True)
        pltpu.async_copy(wb[b], den_sh.at[db[b]], sems[b], add=True)

    def wait_scatter(b):
        pltpu.make_async_copy(rows[b], acc_sh.at[db[b]], sems[b]).wait()
        pltpu.make_async_copy(wb[b], den_sh.at[db[b]], sems[b]).wait()

    def compute(b):
        for vi in range(C // L):
            sv = si[b][pl.ds(vi * L, L)]
            dv = di[b][pl.ds(vi * L, L)]
            e = plsc.load_gather(el_v, [sv]) + plsc.load_gather(er_v, [dv])
            e = jnp.where(e > 0.0, e, 0.2 * e)
            wb[b][pl.ds(vi * L, L)] = jnp.exp(e)
            db[b][pl.ds(vi * L, L)] = dv

        @pl.loop(0, C)
        def _(i):
            iv = jnp.full((L,), i, dtype=jnp.int32)
            wv = plsc.load_gather(wb[b], [iv])
            for j in range(D // L):
                rows[b][i, pl.ds(j * L, L)] = rows[b][i, pl.ds(j * L, L)] * wv

    issue_idx(0, 0)
    issue_idx(1, 1)

    zeros = jnp.zeros((L,), jnp.float32)

    @pl.loop(0, 640, step=L)
    def _(i):
        z_v[pl.ds(i, L)] = zeros

    @pl.loop(0, C)
    def _(i):
        for j in range(D // L):
            rows0[i, pl.ds(j * L, L)] = zeros

    rbase = pl.multiple_of(s * FLUSH_A, 8)

    def _zero_acc(base, nrows):
        for k in range(nrows // C):
            pltpu.sync_copy(rows0, acc_sh.at[pl.ds(base + k * C, C)])
        rem = nrows % C
        if rem:
            pltpu.sync_copy(rows0.at[pl.ds(0, rem)],
                            acc_sh.at[pl.ds(base + nrows - rem, rem)])

    @pl.when(s != NS - 1)
    def _():
        _zero_acc(rbase, FLUSH_A)

    @pl.when(s == NS - 1)
    def _():
        _zero_acc((NS - 1) * FLUSH_A, FLUSH_B)

    dbase = pl.multiple_of(s * 640, 8)
    pltpu.sync_copy(z_v, den_sh.at[pl.ds(dbase, 640)])

    pltpu.make_async_copy(el_hbm, el_v, sem_pre).wait()
    pltpu.make_async_copy(er_hbm, er_v, sem_pre).wait()

    wait_idx(0)
    issue_gather(0)
    plsc.subcore_barrier()

    @pl.loop(0, (NCHUNK - 1) // 2)
    def _(p):
        for b in range(2):
            g = 2 * p + b
            wait_gather(b)

            @pl.when(p + b > 0)
            def _():
                wait_scatter(1 - b)

            wait_idx(1 - b)
            issue_gather(1 - b)
            compute(b)

            @pl.when(g + 2 < NCHUNK)
            def _():
                issue_idx(g + 2, b)

            issue_scatter(b)

    wait_gather(0)
    compute(0)
    wait_scatter(1)
    issue_scatter(0)
    wait_scatter(0)

    plsc.subcore_barrier()

    @pl.when(s != NS - 1)
    def _():
        pltpu.sync_copy(acc_sh.at[pl.ds(rbase, FLUSH_A)],
                        acc_hbm.at[c, pl.ds(rbase, FLUSH_A)])

    @pl.when(s == NS - 1)
    def _():
        pltpu.sync_copy(acc_sh.at[pl.ds((NS - 1) * FLUSH_A, FLUSH_B)],
                        acc_hbm.at[c, pl.ds((NS - 1) * FLUSH_A, FLUSH_B)])

    pltpu.sync_copy(den_sh.at[pl.ds(dbase, 640)],
                    den_hbm.at[c, pl.ds(dbase, 640)])


def _final_body(acc_ref, den_ref, out_ref):
    a = acc_ref[0] + acc_ref[1]
    dsum = den_ref[0] + den_ref[1]
    out_ref[...] = jnp.where(dsum == 0.0, 0.0,
                             a / jnp.where(dsum == 0.0, 1.0, dsum))


def _finalize(acc, den):
    return pl.pallas_call(
        _final_body,
        grid=(N // ROW_BLK,),
        in_specs=[
            pl.BlockSpec((NC, ROW_BLK, D), lambda i: (0, i, 0)),
            pl.BlockSpec((NC, ROW_BLK, 1), lambda i: (0, i, 0)),
        ],
        out_specs=pl.BlockSpec((ROW_BLK, D), lambda i: (i, 0)),
        out_shape=jax.ShapeDtypeStruct((N, D), jnp.float32),
    )(acc, den)


def kernel(features, edge_index, fc_weight, attn_l, attn_r):
    h, el, er = _dense(features, fc_weight, attn_l.T, attn_r.T)
    acc, den = _sc_edges(h, el.reshape(N), er.reshape(N),
                         edge_index.reshape(2 * E))
    return _finalize(acc, den.reshape(NC, DEN_PAD, 1))
